# Initial kernel scaffold; baseline (speedup 1.0000x reference)
import jax
import jax.numpy as jnp
from jax.experimental import pallas as pl


def kernel(sorted_value, sorted_indices, p, k):
    raise NotImplementedError("write your pallas kernel here")


# speedup vs baseline: 2.3810x; 2.3810x over previous
import functools

import jax
import jax.numpy as jnp
from jax import lax
from jax.experimental import pallas as pl
from jax.experimental.pallas import tpu as pltpu
from jax.experimental.pallas import tpu_sc as plsc

L = 16
NEG_INF = float("-inf")


def _scalar_at(ref, idx):
    return ref[pl.ds(idx, L)][0]


@functools.lru_cache(maxsize=None)
def _build(batch: int, vocab: int):
    info = plsc.get_sparse_core_info()
    nc, ns = info.num_cores, info.num_subcores
    nw = nc * ns
    assert batch % nw == 0, (batch, nw)
    rows_per_w = batch // nw
    assert vocab % L == 0 and vocab % 8 == 0
    chunk = 4000
    assert vocab % chunk == 0 and chunk % L == 0
    nchunk = vocab // chunk
    nvreg = vocab // L

    mesh = plsc.VectorSubcoreMesh(core_axis_name="c", subcore_axis_name="s")

    @functools.partial(
        pl.kernel,
        out_type=jax.ShapeDtypeStruct((batch * vocab,), jnp.float32),
        mesh=mesh,
        compiler_params=pltpu.CompilerParams(needs_layout_passes=False),
        scratch_types=[
            pltpu.VMEM((vocab + L,), jnp.float32),
            pltpu.VMEM((chunk,), jnp.int32),
            pltpu.VMEM((chunk,), jnp.float32),
            pltpu.VMEM((batch + L,), jnp.float32),
            pltpu.VMEM((batch + L,), jnp.int32),
        ],
    )
    def sc_kernel(sv_hbm, si_hbm, p_hbm, k_hbm, out_hbm,
                  row_v, idx_v, out_v, p_v, k_v):
        wid = lax.axis_index("s") * nc + lax.axis_index("c")
        pltpu.sync_copy(p_hbm, p_v.at[pl.ds(0, batch)])
        pltpu.sync_copy(k_hbm, k_v.at[pl.ds(0, batch)])

        for r in range(rows_per_w):
            row = wid * rows_per_w + r
            base = row * vocab
            pltpu.sync_copy(sv_hbm.at[pl.ds(base, vocab)],
                            row_v.at[pl.ds(0, vocab)])

            kk = _scalar_at(k_v, row)
            pp = _scalar_at(p_v, row)
            m = row_v[pl.ds(vocab - L, L)][L - 1]

            valid = kk >= 1
            idx_t = jnp.clip(vocab - kk, 0, vocab - 1)
            thresh = _scalar_at(row_v, idx_t)
            lo = jnp.int32(0)
            hi = jnp.int32(vocab)
            for _ in range(17):
                cont = lo < hi
                mid = (lo + hi) // 2
                vm = _scalar_at(row_v, jnp.minimum(mid, vocab - 1))
                below = vm < thresh
                lo = jnp.where(cont & below, mid + 1, lo)
                hi = jnp.where(cont & (~below), mid, hi)
            start = jnp.where(valid, lo, 0)
            g0 = start // L

            def sweep_a(g, acc):
                vv = row_v[pl.ds(g * L, L)]
                jj = lax.iota(jnp.int32, L) + g * L
                e = jnp.where(jj >= start, jnp.exp(vv - m), 0.0)
                return acc + e

            acc = lax.fori_loop(g0, nvreg, sweep_a,
                                jnp.zeros((L,), jnp.float32))
            total = jnp.sum(acc)
            t = (1.0 - pp) * total

            def sweep_b(g, carry):
                s, cnt = carry
                vv = row_v[pl.ds(g * L, L)]
                jj = lax.iota(jnp.int32, L) + g * L
                e = jnp.where(jj >= start, jnp.exp(vv - m), 0.0)
                pc = plsc.cumsum(e) + s
                cond = (pc <= t) & (jj >= start)
                cnt = cnt + jnp.sum(cond.astype(jnp.int32))
                return s + jnp.sum(e), cnt

            _, cnt = lax.fori_loop(g0, nvreg, sweep_b,
                                   (jnp.float32(0.0), jnp.int32(0)))
            c = start + cnt

            def chunk_body(ch, _):
                cbase = base + ch * chunk
                pltpu.sync_copy(si_hbm.at[pl.ds(cbase, chunk)], idx_v)

                def gbody(i, _):
                    idx16 = idx_v[pl.ds(i * L, L)]
                    vals = plsc.load_gather(row_v, [idx16])
                    keep = (idx16 >= c) | (idx16 == vocab - 1)
                    out_v[pl.ds(i * L, L)] = jnp.where(keep, vals, NEG_INF)
                    return 0

                lax.fori_loop(0, chunk // L, gbody, 0)
                pltpu.sync_copy(out_v, out_hbm.at[pl.ds(cbase, chunk)])
                return 0

            lax.fori_loop(0, nchunk, chunk_body, 0)

    return sc_kernel


def kernel(sorted_value, sorted_indices, p, k):
    batch, vocab = sorted_value.shape
    fn = _build(batch, vocab)
    out = fn(sorted_value.reshape(-1),
             sorted_indices.astype(jnp.int32).reshape(-1),
             p.astype(jnp.float32), k.astype(jnp.int32))
    return out.reshape(batch, vocab)


# speedup vs baseline: 3.6503x; 1.5331x over previous
import functools

import jax
import jax.numpy as jnp
from jax import lax
from jax.experimental import pallas as pl
from jax.experimental.pallas import tpu as pltpu
from jax.experimental.pallas import tpu_sc as plsc

L = 16
NEG_INF = float("-inf")


def _scalar_at(ref, idx):
    return ref[pl.ds(idx, L)][0]


@functools.lru_cache(maxsize=None)
def _build(batch: int, vocab: int):
    info = plsc.get_sparse_core_info()
    nc, ns = info.num_cores, info.num_subcores
    nw = nc * ns
    assert batch % nw == 0, (batch, nw)
    rows_per_w = batch // nw
    assert vocab % L == 0 and vocab % 8 == 0
    chunk = 4000
    assert vocab % chunk == 0 and chunk % L == 0
    nchunk = vocab // chunk
    assert nchunk % 2 == 1 and nchunk >= 3
    nvreg = vocab // L

    mesh = plsc.VectorSubcoreMesh(core_axis_name="c", subcore_axis_name="s")

    @functools.partial(
        pl.kernel,
        out_type=jax.ShapeDtypeStruct((batch * vocab,), jnp.float32),
        mesh=mesh,
        compiler_params=pltpu.CompilerParams(needs_layout_passes=False),
        scratch_types=[
            pltpu.VMEM((vocab + L,), jnp.float32),
            pltpu.VMEM((chunk,), jnp.int32),
            pltpu.VMEM((chunk,), jnp.int32),
            pltpu.VMEM((chunk,), jnp.float32),
            pltpu.VMEM((chunk,), jnp.float32),
            pltpu.VMEM((batch + L,), jnp.float32),
            pltpu.VMEM((batch + L,), jnp.int32),
            pltpu.SemaphoreType.DMA,
            pltpu.SemaphoreType.DMA,
            pltpu.SemaphoreType.DMA,
            pltpu.SemaphoreType.DMA,
        ],
    )
    def sc_kernel(sv_hbm, si_hbm, p_hbm, k_hbm, out_hbm,
                  row_v, idx_v0, idx_v1, out_v0, out_v1, p_v, k_v,
                  sem_in0, sem_in1, sem_out0, sem_out1):
        wid = lax.axis_index("s") * nc + lax.axis_index("c")
        pltpu.sync_copy(p_hbm, p_v.at[pl.ds(0, batch)])
        pltpu.sync_copy(k_hbm, k_v.at[pl.ds(0, batch)])

        sem_in = (sem_in0, sem_in1)
        sem_out = (sem_out0, sem_out1)
        idx_v = (idx_v0, idx_v1)
        out_v = (out_v0, out_v1)

        for r in range(rows_per_w):
            row = wid * rows_per_w + r
            base = row * vocab
            pltpu.sync_copy(sv_hbm.at[pl.ds(base, vocab)],
                            row_v.at[pl.ds(0, vocab)])

            def issue_in(ch, b):
                pltpu.async_copy(si_hbm.at[pl.ds(base + ch * chunk, chunk)],
                                 idx_v[b], sem_in[b])

            def wait_in(b):
                pltpu.make_async_copy(si_hbm.at[pl.ds(base, chunk)],
                                      idx_v[b], sem_in[b]).wait()

            def issue_out(ch, b):
                pltpu.async_copy(out_v[b],
                                 out_hbm.at[pl.ds(base + ch * chunk, chunk)],
                                 sem_out[b])

            def wait_out(b):
                pltpu.make_async_copy(out_v[b],
                                      out_hbm.at[pl.ds(base, chunk)],
                                      sem_out[b]).wait()

            issue_in(0, 0)
            issue_in(1, 1)

            kk = _scalar_at(k_v, row)
            pp = _scalar_at(p_v, row)
            m = row_v[pl.ds(vocab - L, L)][L - 1]

            valid = kk >= 1
            idx_t = jnp.clip(vocab - kk, 0, vocab - 1)
            thresh = _scalar_at(row_v, idx_t)
            lo = jnp.int32(0)
            hi = jnp.int32(vocab)
            for _ in range(17):
                cont = lo < hi
                mid = (lo + hi) // 2
                vm = _scalar_at(row_v, jnp.minimum(mid, vocab - 1))
                below = vm < thresh
                lo = jnp.where(cont & below, mid + 1, lo)
                hi = jnp.where(cont & (~below), mid, hi)
            start = jnp.where(valid, lo, 0)
            g0 = start // L

            def sweep_a(g, acc):
                vv = row_v[pl.ds(g * L, L)]
                jj = lax.iota(jnp.int32, L) + g * L
                e = jnp.where(jj >= start, jnp.exp(vv - m), 0.0)
                return acc + e

            acc = lax.fori_loop(g0, nvreg, sweep_a,
                                jnp.zeros((L,), jnp.float32))
            total = jnp.sum(acc)
            t = (1.0 - pp) * total

            def sweep_b(g, carry):
                s, cnt = carry
                vv = row_v[pl.ds(g * L, L)]
                jj = lax.iota(jnp.int32, L) + g * L
                e = jnp.where(jj >= start, jnp.exp(vv - m), 0.0)
                pc = plsc.cumsum(e) + s
                cond = (pc <= t) & (jj >= start)
                cnt = cnt + jnp.sum(cond.astype(jnp.int32))
                return s + jnp.sum(e), cnt

            _, cnt = lax.fori_loop(g0, nvreg, sweep_b,
                                   (jnp.float32(0.0), jnp.int32(0)))
            c = start + cnt

            def gather_chunk(ch, b):
                @plsc.parallel_loop(0, chunk, step=L, unroll=8)
                def gbody(i):
                    idx16 = idx_v[b][pl.ds(i, L)]
                    vals = plsc.load_gather(row_v, [idx16])
                    keep = (idx16 >= c) | (idx16 == vocab - 1)
                    out_v[b][pl.ds(i, L)] = jnp.where(keep, vals, NEG_INF)

                issue_out(ch, b)

            wait_in(0)
            gather_chunk(0, 0)
            issue_in(2, 0)
            wait_in(1)
            gather_chunk(1, 1)
            issue_in(3, 1)

            def pair_body(t, _):
                wait_in(0)
                wait_out(0)
                gather_chunk(2 * t, 0)
                issue_in(2 * t + 2, 0)

                wait_in(1)
                wait_out(1)
                gather_chunk(2 * t + 1, 1)

                @pl.when(2 * t + 3 < nchunk)
                def _():
                    issue_in(2 * t + 3, 1)

                return 0

            lax.fori_loop(1, (nchunk - 1) // 2, pair_body, 0)

            wait_in(0)
            wait_out(0)
            gather_chunk(nchunk - 1, 0)

            wait_out(0)
            wait_out(1)

    return sc_kernel


def kernel(sorted_value, sorted_indices, p, k):
    batch, vocab = sorted_value.shape
    fn = _build(batch, vocab)
    out = fn(sorted_value.reshape(-1),
             sorted_indices.astype(jnp.int32).reshape(-1),
             p.astype(jnp.float32), k.astype(jnp.int32))
    return out.reshape(batch, vocab)
